# baseline (speedup 1.0000x reference)
import functools

import jax
import jax.numpy as jnp
from jax import lax
from jax.experimental import pallas as pl
from jax.experimental.pallas import tpu as pltpu
from jax.experimental.pallas import tpu_sc as plsc

B = 4096
D = 2048
S = 4096

NC = 2
NS = 16
NW = NC * NS
ROWS_PER_W = B // NW
R_CHUNK = 8
N_CHUNK = ROWS_PER_W // R_CHUNK

_MIN32 = -2147483648


def _sc_body(x_hbm, h1_hbm, h2_hbm, h3_hbm, s1_hbm, s2_hbm, s3_hbm, out_hbm,
             c1, c2, c3, tmp_s, a1, a2, a3,
             xb0, xb1, ob0, ob1,
             in_sem0, in_sem1, out_sem0, out_sem1):
    wid = lax.axis_index("c") * NS + lax.axis_index("s")
    row0 = wid * ROWS_PER_W

    pltpu.sync_copy(h1_hbm, c1)
    pltpu.sync_copy(h2_hbm, c2)
    pltpu.sync_copy(h3_hbm, c3)
    for s_hbm, cref in ((s1_hbm, c1), (s2_hbm, c2), (s3_hbm, c3)):
        pltpu.sync_copy(s_hbm, tmp_s)

        def pack(j, _, cref=cref):
            dsl = pl.ds(j * 16, 16)
            sbit = jnp.where(tmp_s[dsl] < 0.0,
                             jnp.full((16,), _MIN32, jnp.int32),
                             jnp.zeros((16,), jnp.int32))
            cref[dsl] = jnp.bitwise_or(cref[dsl], sbit)
            return 0

        lax.fori_loop(0, D // 16, pack, 0)

    def zero(k, _):
        dsl = pl.ds(k * 16, 16)
        z = jnp.zeros((16,), jnp.float32)
        a1[dsl] = z
        a2[dsl] = z
        a3[dsl] = z
        return 0

    lax.fori_loop(0, S // 16, zero, 0)

    def compute(xb, ob):
        def row_body(r, _):
            def scat(j, _):
                dsl = pl.ds(j * 16, 16)
                xv = xb[r, dsl]
                for cref, accref in ((c1, a1), (c2, a2), (c3, a3)):
                    cv = cref[dsl]
                    idx = jnp.bitwise_and(cv, S - 1)
                    val = xv * jnp.where(cv < 0,
                                         jnp.full((16,), -1.0, jnp.float32),
                                         jnp.full((16,), 1.0, jnp.float32))
                    plsc.addupdate_scatter(accref, [idx], val)
                return 0

            lax.fori_loop(0, D // 16, scat, 0)

            def prod(k, _):
                dsl = pl.ds(k * 16, 16)
                ob[r, dsl] = a1[dsl] * a2[dsl] * a3[dsl]
                z = jnp.zeros((16,), jnp.float32)
                a1[dsl] = z
                a2[dsl] = z
                a3[dsl] = z
                return 0

            lax.fori_loop(0, S // 16, prod, 0)
            return 0

        lax.fori_loop(0, R_CHUNK, row_body, 0)

    xbufs = (xb0, xb1)
    obufs = (ob0, ob1)
    in_sems = (in_sem0, in_sem1)
    out_sems = (out_sem0, out_sem1)

    def start_in(g):
        return pltpu.async_copy(
            x_hbm.at[pl.ds(row0 + g * R_CHUNK, R_CHUNK)],
            xbufs[g % 2], in_sems[g % 2])

    def start_out(g):
        return pltpu.async_copy(
            obufs[g % 2],
            out_hbm.at[pl.ds(row0 + g * R_CHUNK, R_CHUNK)],
            out_sems[g % 2])

    h_in = [None, None]
    h_out = [None, None]
    h_in[0] = start_in(0)
    for g in range(N_CHUNK):
        if g + 1 < N_CHUNK:
            h_in[(g + 1) % 2] = start_in(g + 1)
        h_in[g % 2].wait()
        if h_out[g % 2] is not None:
            h_out[g % 2].wait()
        compute(xbufs[g % 2], obufs[g % 2])
        h_out[g % 2] = start_out(g)
    h_out[(N_CHUNK - 2) % 2].wait()
    h_out[(N_CHUNK - 1) % 2].wait()


def _tensor_sketch_sc(x, hash1, hash2, hash3, sign1, sign2, sign3):
    mesh = plsc.VectorSubcoreMesh(core_axis_name="c", subcore_axis_name="s")
    k = functools.partial(
        pl.kernel, mesh=mesh,
        out_type=jax.ShapeDtypeStruct((B, S), jnp.float32),
        compiler_params=pltpu.CompilerParams(needs_layout_passes=False),
        scratch_types=[
            pltpu.VMEM((D,), jnp.int32),
            pltpu.VMEM((D,), jnp.int32),
            pltpu.VMEM((D,), jnp.int32),
            pltpu.VMEM((D,), jnp.float32),
            pltpu.VMEM((S,), jnp.float32),
            pltpu.VMEM((S,), jnp.float32),
            pltpu.VMEM((S,), jnp.float32),
            pltpu.VMEM((R_CHUNK, D), jnp.float32),
            pltpu.VMEM((R_CHUNK, D), jnp.float32),
            pltpu.VMEM((R_CHUNK, S), jnp.float32),
            pltpu.VMEM((R_CHUNK, S), jnp.float32),
            pltpu.SemaphoreType.DMA,
            pltpu.SemaphoreType.DMA,
            pltpu.SemaphoreType.DMA,
            pltpu.SemaphoreType.DMA,
        ],
    )(_sc_body)
    return k(x, hash1, hash2, hash3, sign1, sign2, sign3)


@jax.jit
def kernel(x, sign1, sign2, sign3, hash1, hash2, hash3):
    return _tensor_sketch_sc(x, hash1, hash2, hash3, sign1, sign2, sign3)


# speedup vs baseline: 3.0836x; 3.0836x over previous
import functools

import jax
import jax.numpy as jnp
from jax import lax
from jax.experimental import pallas as pl
from jax.experimental.pallas import tpu as pltpu
from jax.experimental.pallas import tpu_sc as plsc

B = 4096
D = 2048
S = 4096

NC = 2
NS = 16
NW = NC * NS
ROWS_PER_W = B // NW
R_CHUNK = 8
N_CHUNK = ROWS_PER_W // R_CHUNK

_MIN32 = -2147483648


def _sc_body(x_hbm, h1_hbm, h2_hbm, h3_hbm, s1_hbm, s2_hbm, s3_hbm, out_hbm,
             c1, c2, c3, tmp_s, a1, a2, a3,
             xb0, xb1, ob0, ob1,
             in_sem0, in_sem1, out_sem0, out_sem1):
    wid = lax.axis_index("c") * NS + lax.axis_index("s")
    row0 = wid * ROWS_PER_W

    pltpu.sync_copy(h1_hbm, c1)
    pltpu.sync_copy(h2_hbm, c2)
    pltpu.sync_copy(h3_hbm, c3)
    for s_hbm, cref in ((s1_hbm, c1), (s2_hbm, c2), (s3_hbm, c3)):
        pltpu.sync_copy(s_hbm, tmp_s)

        def pack(j, _, cref=cref):
            dsl = pl.ds(j * 16, 16)
            sbit = jnp.where(tmp_s[dsl] < 0.0,
                             jnp.full((16,), _MIN32, jnp.int32),
                             jnp.zeros((16,), jnp.int32))
            cref[dsl] = jnp.bitwise_or(cref[dsl], sbit)
            return 0

        lax.fori_loop(0, D // 16, pack, 0)

    def zero(k, _):
        dsl = pl.ds(k * 16, 16)
        z = jnp.zeros((16,), jnp.float32)
        a1[dsl] = z
        a2[dsl] = z
        a3[dsl] = z
        return 0

    lax.fori_loop(0, S // 16, zero, 0)

    def compute(xb, ob):
        def row_body(r, _):
            @plsc.parallel_loop(0, D // 16, unroll=8)
            def scat(j):
                dsl = pl.ds(j * 16, 16)
                xv = xb[r, dsl]
                for cref, accref in ((c1, a1), (c2, a2), (c3, a3)):
                    cv = cref[dsl]
                    idx = jnp.bitwise_and(cv, S - 1)
                    val = xv * jnp.where(cv < 0,
                                         jnp.full((16,), -1.0, jnp.float32),
                                         jnp.full((16,), 1.0, jnp.float32))
                    plsc.addupdate_scatter(accref, [idx], val)

            @plsc.parallel_loop(0, S // 16, unroll=8)
            def prod(k):
                dsl = pl.ds(k * 16, 16)
                ob[r, dsl] = a1[dsl] * a2[dsl] * a3[dsl]
                z = jnp.zeros((16,), jnp.float32)
                a1[dsl] = z
                a2[dsl] = z
                a3[dsl] = z
            return 0

        lax.fori_loop(0, R_CHUNK, row_body, 0)

    xbufs = (xb0, xb1)
    obufs = (ob0, ob1)
    in_sems = (in_sem0, in_sem1)
    out_sems = (out_sem0, out_sem1)

    def start_in(g):
        return pltpu.async_copy(
            x_hbm.at[pl.ds(row0 + g * R_CHUNK, R_CHUNK)],
            xbufs[g % 2], in_sems[g % 2])

    def start_out(g):
        return pltpu.async_copy(
            obufs[g % 2],
            out_hbm.at[pl.ds(row0 + g * R_CHUNK, R_CHUNK)],
            out_sems[g % 2])

    h_in = [None, None]
    h_out = [None, None]
    h_in[0] = start_in(0)
    for g in range(N_CHUNK):
        if g + 1 < N_CHUNK:
            h_in[(g + 1) % 2] = start_in(g + 1)
        h_in[g % 2].wait()
        if h_out[g % 2] is not None:
            h_out[g % 2].wait()
        compute(xbufs[g % 2], obufs[g % 2])
        h_out[g % 2] = start_out(g)
    h_out[(N_CHUNK - 2) % 2].wait()
    h_out[(N_CHUNK - 1) % 2].wait()


def _tensor_sketch_sc(x, hash1, hash2, hash3, sign1, sign2, sign3):
    mesh = plsc.VectorSubcoreMesh(core_axis_name="c", subcore_axis_name="s")
    k = functools.partial(
        pl.kernel, mesh=mesh,
        out_type=jax.ShapeDtypeStruct((B, S), jnp.float32),
        compiler_params=pltpu.CompilerParams(needs_layout_passes=False),
        scratch_types=[
            pltpu.VMEM((D,), jnp.int32),
            pltpu.VMEM((D,), jnp.int32),
            pltpu.VMEM((D,), jnp.int32),
            pltpu.VMEM((D,), jnp.float32),
            pltpu.VMEM((S,), jnp.float32),
            pltpu.VMEM((S,), jnp.float32),
            pltpu.VMEM((S,), jnp.float32),
            pltpu.VMEM((R_CHUNK, D), jnp.float32),
            pltpu.VMEM((R_CHUNK, D), jnp.float32),
            pltpu.VMEM((R_CHUNK, S), jnp.float32),
            pltpu.VMEM((R_CHUNK, S), jnp.float32),
            pltpu.SemaphoreType.DMA,
            pltpu.SemaphoreType.DMA,
            pltpu.SemaphoreType.DMA,
            pltpu.SemaphoreType.DMA,
        ],
    )(_sc_body)
    return k(x, hash1, hash2, hash3, sign1, sign2, sign3)


@jax.jit
def kernel(x, sign1, sign2, sign3, hash1, hash2, hash3):
    return _tensor_sketch_sc(x, hash1, hash2, hash3, sign1, sign2, sign3)


# speedup vs baseline: 3.3561x; 1.0884x over previous
import functools

import jax
import jax.numpy as jnp
from jax import lax
from jax.experimental import pallas as pl
from jax.experimental.pallas import tpu as pltpu
from jax.experimental.pallas import tpu_sc as plsc

B = 4096
D = 2048
S = 4096

NC = 2
NS = 16
NW = NC * NS
R_CHUNK = 8

_MIN32 = -2147483648

B_TC = 1792
B_SC = B - B_TC


def _sc_body(rows_per_w, x_hbm, h1_hbm, h2_hbm, h3_hbm, s1_hbm, s2_hbm,
             s3_hbm, out_hbm,
             c1, c2, c3, tmp_s, a1, a2, a3,
             xb0, xb1, ob0, ob1,
             in_sem0, in_sem1, out_sem0, out_sem1):
    wid = lax.axis_index("c") * NS + lax.axis_index("s")
    row0 = wid * rows_per_w
    n_chunk = rows_per_w // R_CHUNK

    pltpu.sync_copy(h1_hbm, c1)
    pltpu.sync_copy(h2_hbm, c2)
    pltpu.sync_copy(h3_hbm, c3)
    for s_hbm, cref in ((s1_hbm, c1), (s2_hbm, c2), (s3_hbm, c3)):
        pltpu.sync_copy(s_hbm, tmp_s)

        @plsc.parallel_loop(0, D // 16, unroll=8)
        def pack(j, cref=cref):
            dsl = pl.ds(j * 16, 16)
            sbit = jnp.where(tmp_s[dsl] < 0.0,
                             jnp.full((16,), _MIN32, jnp.int32),
                             jnp.zeros((16,), jnp.int32))
            cref[dsl] = jnp.bitwise_or(cref[dsl], sbit)

    @plsc.parallel_loop(0, S // 16, unroll=8)
    def zero(k):
        dsl = pl.ds(k * 16, 16)
        z = jnp.zeros((16,), jnp.float32)
        a1[dsl] = z
        a2[dsl] = z
        a3[dsl] = z

    def compute(xb, ob):
        def row_body(r, _):
            @plsc.parallel_loop(0, D // 16, unroll=8)
            def scat(j):
                dsl = pl.ds(j * 16, 16)
                xv = xb[r, dsl]
                for cref, accref in ((c1, a1), (c2, a2), (c3, a3)):
                    cv = cref[dsl]
                    idx = jnp.bitwise_and(cv, S - 1)
                    val = xv * jnp.where(cv < 0,
                                         jnp.full((16,), -1.0, jnp.float32),
                                         jnp.full((16,), 1.0, jnp.float32))
                    plsc.addupdate_scatter(accref, [idx], val)

            @plsc.parallel_loop(0, S // 16, unroll=8)
            def prod(k):
                dsl = pl.ds(k * 16, 16)
                ob[r, dsl] = a1[dsl] * a2[dsl] * a3[dsl]
                z = jnp.zeros((16,), jnp.float32)
                a1[dsl] = z
                a2[dsl] = z
                a3[dsl] = z
            return 0

        lax.fori_loop(0, R_CHUNK, row_body, 0)

    xbufs = (xb0, xb1)
    obufs = (ob0, ob1)
    in_sems = (in_sem0, in_sem1)
    out_sems = (out_sem0, out_sem1)

    def start_in(g):
        return pltpu.async_copy(
            x_hbm.at[pl.ds(row0 + g * R_CHUNK, R_CHUNK)],
            xbufs[g % 2], in_sems[g % 2])

    def start_out(g):
        return pltpu.async_copy(
            obufs[g % 2],
            out_hbm.at[pl.ds(row0 + g * R_CHUNK, R_CHUNK)],
            out_sems[g % 2])

    h_in = [None, None]
    h_out = [None, None]
    h_in[0] = start_in(0)
    for g in range(n_chunk):
        if g + 1 < n_chunk:
            h_in[(g + 1) % 2] = start_in(g + 1)
        h_in[g % 2].wait()
        if h_out[g % 2] is not None:
            h_out[g % 2].wait()
        compute(xbufs[g % 2], obufs[g % 2])
        h_out[g % 2] = start_out(g)
    h_out[(n_chunk - 2) % 2].wait()
    h_out[(n_chunk - 1) % 2].wait()


def _tensor_sketch_sc(x, hash1, hash2, hash3, sign1, sign2, sign3):
    rows = x.shape[0]
    rows_per_w = rows // NW
    mesh = plsc.VectorSubcoreMesh(core_axis_name="c", subcore_axis_name="s")
    k = functools.partial(
        pl.kernel, mesh=mesh,
        out_type=jax.ShapeDtypeStruct((rows, S), jnp.float32),
        compiler_params=pltpu.CompilerParams(needs_layout_passes=False),
        scratch_types=[
            pltpu.VMEM((D,), jnp.int32),
            pltpu.VMEM((D,), jnp.int32),
            pltpu.VMEM((D,), jnp.int32),
            pltpu.VMEM((D,), jnp.float32),
            pltpu.VMEM((S,), jnp.float32),
            pltpu.VMEM((S,), jnp.float32),
            pltpu.VMEM((S,), jnp.float32),
            pltpu.VMEM((R_CHUNK, D), jnp.float32),
            pltpu.VMEM((R_CHUNK, D), jnp.float32),
            pltpu.VMEM((R_CHUNK, S), jnp.float32),
            pltpu.VMEM((R_CHUNK, S), jnp.float32),
            pltpu.SemaphoreType.DMA,
            pltpu.SemaphoreType.DMA,
            pltpu.SemaphoreType.DMA,
            pltpu.SemaphoreType.DMA,
        ],
    )(functools.partial(_sc_body, rows_per_w))
    return k(x, hash1, hash2, hash3, sign1, sign2, sign3)


TC_B_BLK = 256
TC_S_BLK = 1024


def _tc_body(x_ref, h1_ref, h2_ref, h3_ref, s1_ref, s2_ref, s3_ref, out_ref,
             m1_ref, m2_ref, m3_ref):
    s_idx = pl.program_id(0)
    b_idx = pl.program_id(1)

    @pl.when(b_idx == 0)
    def _build_onehots():
        col = jax.lax.broadcasted_iota(jnp.int32, (D, TC_S_BLK), 1) + s_idx * TC_S_BLK
        zero = jnp.zeros((D, TC_S_BLK), dtype=jnp.float32)
        m1_ref[...] = jnp.where(h1_ref[...] == col, s1_ref[...], zero).astype(jnp.bfloat16)
        m2_ref[...] = jnp.where(h2_ref[...] == col, s2_ref[...], zero).astype(jnp.bfloat16)
        m3_ref[...] = jnp.where(h3_ref[...] == col, s3_ref[...], zero).astype(jnp.bfloat16)

    xb = x_ref[...]
    a1 = jnp.dot(xb, m1_ref[...], preferred_element_type=jnp.float32)
    a2 = jnp.dot(xb, m2_ref[...], preferred_element_type=jnp.float32)
    a3 = jnp.dot(xb, m3_ref[...], preferred_element_type=jnp.float32)
    out_ref[...] = a1 * a2 * a3


def _tensor_sketch_tc(x, hash1, hash2, hash3, sign1, sign2, sign3):
    rows = x.shape[0]
    x16 = x.astype(jnp.bfloat16)
    h1 = hash1.reshape(D, 1)
    h2 = hash2.reshape(D, 1)
    h3 = hash3.reshape(D, 1)
    s1 = sign1.reshape(D, 1)
    s2 = sign2.reshape(D, 1)
    s3 = sign3.reshape(D, 1)

    full = lambda: pl.BlockSpec((D, 1), lambda s, b: (0, 0))
    return pl.pallas_call(
        _tc_body,
        grid=(S // TC_S_BLK, rows // TC_B_BLK),
        in_specs=[
            pl.BlockSpec((TC_B_BLK, D), lambda s, b: (b, 0)),
            full(), full(), full(), full(), full(), full(),
        ],
        out_specs=pl.BlockSpec((TC_B_BLK, TC_S_BLK), lambda s, b: (b, s)),
        out_shape=jax.ShapeDtypeStruct((rows, S), jnp.float32),
        scratch_shapes=[
            pltpu.VMEM((D, TC_S_BLK), jnp.bfloat16),
            pltpu.VMEM((D, TC_S_BLK), jnp.bfloat16),
            pltpu.VMEM((D, TC_S_BLK), jnp.bfloat16),
        ],
        compiler_params=pltpu.CompilerParams(
            dimension_semantics=("arbitrary", "arbitrary"),
        ),
    )(x16, h1, h2, h3, s1, s2, s3)


@jax.jit
def kernel(x, sign1, sign2, sign3, hash1, hash2, hash3):
    out_tc = _tensor_sketch_tc(x[:B_TC], hash1, hash2, hash3, sign1, sign2, sign3)
    out_sc = _tensor_sketch_sc(x[B_TC:], hash1, hash2, hash3, sign1, sign2, sign3)
    return jnp.concatenate([out_tc, out_sc], axis=0)


# speedup vs baseline: 5.3755x; 1.6017x over previous
import functools

import jax
import jax.numpy as jnp
from jax import lax
from jax.experimental import pallas as pl
from jax.experimental.pallas import tpu as pltpu
from jax.experimental.pallas import tpu_sc as plsc

B = 4096
D = 2048
S = 4096

NC = 2
NS = 16
NW = NC * NS
R_CHUNK = 8

_MIN32 = -2147483648
A_PAD = S + 16
L_PAD = D + 16


def _sc_body(rows_per_w, x_hbm, h1_hbm, h2_hbm, h3_hbm, s1_hbm, s2_hbm,
             s3_hbm, out_hbm,
             c1, c2, c3, a1, a2, a3,
             kl1, kl2, kl3, dl1, dl2, dl3,
             xb0, xb1, ob0, ob1,
             in_sem0, in_sem1, out_sem0, out_sem1):
    wid = lax.axis_index("c") * NS + lax.axis_index("s")
    row0 = wid * rows_per_w
    n_chunk = rows_per_w // R_CHUNK

    zeros16 = jnp.zeros((16,), jnp.float32)
    ones16 = jnp.ones((16,), jnp.float32)

    pltpu.sync_copy(h1_hbm, c1)
    pltpu.sync_copy(h2_hbm, c2)
    pltpu.sync_copy(h3_hbm, c3)
    for s_hbm, cref in ((s1_hbm, c1), (s2_hbm, c2), (s3_hbm, c3)):
        pltpu.sync_copy(s_hbm, xb0.at[0])

        @plsc.parallel_loop(0, D // 16, unroll=8)
        def pack(j, cref=cref):
            dsl = pl.ds(j * 16, 16)
            sbit = jnp.where(xb0[0, dsl] < 0.0,
                             jnp.full((16,), _MIN32, jnp.int32),
                             jnp.zeros((16,), jnp.int32))
            cref[dsl] = jnp.bitwise_or(cref[dsl], sbit)

    @plsc.parallel_loop(0, A_PAD // 16, unroll=8)
    def zero_acc(k):
        dsl = pl.ds(k * 16, 16)
        a1[dsl] = zeros16
        a2[dsl] = zeros16
        a3[dsl] = zeros16

    for cref, accref in ((c1, a1), (c2, a2), (c3, a3)):
        @plsc.parallel_loop(0, D // 16, unroll=8)
        def touch(j, cref=cref, accref=accref):
            dsl = pl.ds(j * 16, 16)
            idx = jnp.bitwise_and(cref[dsl], S - 1)
            plsc.store_scatter(accref, [idx], ones16)

    iota16 = lax.iota(jnp.int32, 16)

    def build_list(cref, klref, dlref):
        def step(j, off):
            dsl = pl.ds(j * 16, 16)
            cv = cref[dsl]
            idx = jnp.bitwise_and(cv, S - 1)
            t1 = plsc.load_gather(a1, [idx])
            t2 = plsc.load_gather(a2, [idx])
            t3 = plsc.load_gather(a3, [idx])
            keep = (t1 * t2 * t3) > 0.5
            plsc.store_compressed(klref.at[pl.ds(off, 16)], cv, mask=keep)
            plsc.store_compressed(dlref.at[pl.ds(off, 16)], j * 16 + iota16,
                                  mask=keep)
            cnt = jnp.max(plsc.all_reduce_population_count(keep))
            return off + cnt

        nk = lax.fori_loop(0, D // 16, step, jnp.int32(0))
        klref[pl.ds(nk, 16)] = jnp.full((16,), S, jnp.int32)
        dlref[pl.ds(nk, 16)] = jnp.zeros((16,), jnp.int32)
        return (nk + 15) >> 4

    nt1 = build_list(c1, kl1, dl1)
    nt2 = build_list(c2, kl2, dl2)
    nt3 = build_list(c3, kl3, dl3)

    for cref, accref in ((c1, a1), (c2, a2), (c3, a3)):
        @plsc.parallel_loop(0, D // 16, unroll=8)
        def untouch(j, cref=cref, accref=accref):
            dsl = pl.ds(j * 16, 16)
            idx = jnp.bitwise_and(cref[dsl], S - 1)
            plsc.store_scatter(accref, [idx], zeros16)

    for ob in (ob0, ob1):
        def zrow(r, _, ob=ob):
            @plsc.parallel_loop(0, S // 16, unroll=4)
            def zero_ob(k):
                ob[r, pl.ds(k * 16, 16)] = zeros16
            return 0

        lax.fori_loop(0, R_CHUNK, zrow, 0)

    bmask = 8191

    def compute(xb, ob):
        def row_body(r, _):
            rvec = jnp.full((16,), r, jnp.int32)
            for klref, dlref, accref, nt in ((kl1, dl1, a1, nt1),
                                             (kl2, dl2, a2, nt2),
                                             (kl3, dl3, a3, nt3)):
                @plsc.parallel_loop(0, nt, unroll=4)
                def scat(j, klref=klref, dlref=dlref, accref=accref,
                         rvec=rvec):
                    dsl = pl.ds(j * 16, 16)
                    cv = klref[dsl]
                    dv = dlref[dsl]
                    xv = plsc.load_gather(xb, [rvec, dv])
                    idx = jnp.bitwise_and(cv, bmask)
                    val = xv * jnp.where(cv < 0,
                                         jnp.full((16,), -1.0, jnp.float32),
                                         jnp.full((16,), 1.0, jnp.float32))
                    plsc.addupdate_scatter(accref, [idx], val)

            @plsc.parallel_loop(0, nt1, unroll=4)
            def prod(j, rvec=rvec):
                dsl = pl.ds(j * 16, 16)
                bv = jnp.bitwise_and(kl1[dsl], bmask)
                p = (plsc.load_gather(a1, [bv])
                     * plsc.load_gather(a2, [bv])
                     * plsc.load_gather(a3, [bv]))
                plsc.store_scatter(ob, [rvec, bv], p, mask=bv < S)

            for klref, accref, nt in ((kl1, a1, nt1), (kl2, a2, nt2),
                                      (kl3, a3, nt3)):
                @plsc.parallel_loop(0, nt, unroll=4)
                def rezero(j, klref=klref, accref=accref):
                    dsl = pl.ds(j * 16, 16)
                    idx = jnp.bitwise_and(klref[dsl], bmask)
                    plsc.store_scatter(accref, [idx], zeros16)
            return 0

        lax.fori_loop(0, R_CHUNK, row_body, 0)

    def clear_ob(ob):
        def row_body(r, _):
            rvec = jnp.full((16,), r, jnp.int32)

            @plsc.parallel_loop(0, nt1, unroll=4)
            def zb(j, rvec=rvec):
                dsl = pl.ds(j * 16, 16)
                bv = jnp.bitwise_and(kl1[dsl], bmask)
                plsc.store_scatter(ob, [rvec, bv], zeros16, mask=bv < S)
            return 0

        lax.fori_loop(0, R_CHUNK, row_body, 0)

    bufs = ((xb0, ob0, in_sem0, out_sem0), (xb1, ob1, in_sem1, out_sem1))

    def in_copy(g, xb, isem):
        return pltpu.make_async_copy(
            x_hbm.at[pl.ds(row0 + g * R_CHUNK, R_CHUNK)], xb, isem)

    def out_copy(g, ob, osem):
        return pltpu.make_async_copy(
            ob, out_hbm.at[pl.ds(row0 + g * R_CHUNK, R_CHUNK)], osem)

    in_copy(0, xb0, in_sem0).start()
    in_copy(1, xb1, in_sem1).start()

    def pair_body(p, _):
        for b in range(2):
            xb, ob, isem, osem = bufs[b]
            g = 2 * p + b
            in_copy(g, xb, isem).wait()

            @pl.when(p > 0)
            def _drain_out():
                out_copy(g - 2, ob, osem).wait()
                clear_ob(ob)

            compute(xb, ob)
            out_copy(g, ob, osem).start()

            @pl.when(g + 2 < n_chunk)
            def _prefetch():
                in_copy(g + 2, xb, isem).start()
        return 0

    lax.fori_loop(0, n_chunk // 2, pair_body, 0)
    out_copy(n_chunk - 2, ob0, out_sem0).wait()
    out_copy(n_chunk - 1, ob1, out_sem1).wait()


def _tensor_sketch_sc(x, hash1, hash2, hash3, sign1, sign2, sign3):
    rows = x.shape[0]
    rows_per_w = rows // NW
    mesh = plsc.VectorSubcoreMesh(core_axis_name="c", subcore_axis_name="s")
    k = functools.partial(
        pl.kernel, mesh=mesh,
        out_type=jax.ShapeDtypeStruct((rows, S), jnp.float32),
        compiler_params=pltpu.CompilerParams(needs_layout_passes=False),
        scratch_types=[
            pltpu.VMEM((D,), jnp.int32),
            pltpu.VMEM((D,), jnp.int32),
            pltpu.VMEM((D,), jnp.int32),
            pltpu.VMEM((A_PAD,), jnp.float32),
            pltpu.VMEM((A_PAD,), jnp.float32),
            pltpu.VMEM((A_PAD,), jnp.float32),
            pltpu.VMEM((L_PAD,), jnp.int32),
            pltpu.VMEM((L_PAD,), jnp.int32),
            pltpu.VMEM((L_PAD,), jnp.int32),
            pltpu.VMEM((L_PAD,), jnp.int32),
            pltpu.VMEM((L_PAD,), jnp.int32),
            pltpu.VMEM((L_PAD,), jnp.int32),
            pltpu.VMEM((R_CHUNK, D), jnp.float32),
            pltpu.VMEM((R_CHUNK, D), jnp.float32),
            pltpu.VMEM((R_CHUNK, S), jnp.float32),
            pltpu.VMEM((R_CHUNK, S), jnp.float32),
            pltpu.SemaphoreType.DMA,
            pltpu.SemaphoreType.DMA,
            pltpu.SemaphoreType.DMA,
            pltpu.SemaphoreType.DMA,
        ],
    )(functools.partial(_sc_body, rows_per_w))
    return k(x, hash1, hash2, hash3, sign1, sign2, sign3)


@jax.jit
def kernel(x, sign1, sign2, sign3, hash1, hash2, hash3):
    return _tensor_sketch_sc(x, hash1, hash2, hash3, sign1, sign2, sign3)
